# baseline (speedup 1.0000x reference)
import functools

import jax
import jax.numpy as jnp
from jax.experimental import pallas as pl
from jax.experimental.pallas import tpu as pltpu


def _mlp_kernel(x_ref, w1_ref, b1_ref, w2_ref, b2_ref, mu_ref):
    xb = x_ref[...].astype(jnp.bfloat16)
    h = jnp.dot(xb, w1_ref[...], preferred_element_type=jnp.float32)
    h = jnp.maximum(h + b1_ref[...], 0.0)
    mu = jnp.dot(h.astype(jnp.bfloat16), w2_ref[...],
                 preferred_element_type=jnp.float32)
    mu_ref[...] = mu + b2_ref[...]


def _round_up(n, m):
    return ((n + m - 1) // m) * m


@functools.partial(jax.jit, static_argnames=("batch_tile",))
def _forward(x, w1, b1, w2, b2, batch_tile=2048):
    B, S = x.shape
    H = w1.shape[1]
    A = w2.shape[1]

    TB = min(batch_tile, _round_up(B, 8))
    Bp = _round_up(B, TB)
    x_p = x if Bp == B else jnp.pad(x, ((0, Bp - B), (0, 0)))
    n_tiles = Bp // TB

    mu_p = pl.pallas_call(
        _mlp_kernel,
        out_shape=jax.ShapeDtypeStruct((Bp, A), jnp.float32),
        grid=(n_tiles,),
        in_specs=[
            pl.BlockSpec((TB, S), lambda i: (i, 0)),
            pl.BlockSpec((S, H), lambda i: (0, 0)),
            pl.BlockSpec((1, H), lambda i: (0, 0)),
            pl.BlockSpec((H, A), lambda i: (0, 0)),
            pl.BlockSpec((1, A), lambda i: (0, 0)),
        ],
        out_specs=pl.BlockSpec((TB, A), lambda i: (i, 0)),
        compiler_params=pltpu.CompilerParams(
            dimension_semantics=("parallel",)),
    )(x_p, w1.astype(jnp.bfloat16), b1, w2.astype(jnp.bfloat16), b2)
    return mu_p if Bp == B else mu_p[:B]


def kernel(x, w1, b1, w2, b2, sigma_param, episode_number):
    mu = _forward(x, w1, b1, w2, b2)
    sigma = jnp.asarray(5.0, dtype=jnp.float32)
    return mu, sigma


# speedup vs baseline: 1.3645x; 1.3645x over previous
import functools

import jax
import jax.numpy as jnp
from jax.experimental import pallas as pl
from jax.experimental.pallas import tpu as pltpu


def _mlp_kernel(x_ref, w1_ref, b1_ref, w2_ref, b2_ref, mu_ref):
    xb = x_ref[...].astype(jnp.bfloat16)
    w1b = w1_ref[...].astype(jnp.bfloat16)
    h = jnp.dot(xb, w1b, preferred_element_type=jnp.float32)
    h = jnp.maximum(h + b1_ref[...], 0.0)
    w2b = w2_ref[...].astype(jnp.bfloat16)
    mu = jnp.dot(h.astype(jnp.bfloat16), w2b,
                 preferred_element_type=jnp.float32)
    mu_ref[...] = mu + b2_ref[...]


def _round_up(n, m):
    return ((n + m - 1) // m) * m


@functools.partial(jax.jit, static_argnames=("batch_tile",))
def _forward(x, w1, b1, w2, b2, batch_tile=4096):
    B, S = x.shape
    H = w1.shape[1]
    A = w2.shape[1]

    TB = min(batch_tile, _round_up(B, 8))
    Bp = _round_up(B, TB)
    x_p = x if Bp == B else jnp.pad(x, ((0, Bp - B), (0, 0)))
    n_tiles = Bp // TB

    mu_p = pl.pallas_call(
        _mlp_kernel,
        out_shape=jax.ShapeDtypeStruct((Bp, A), jnp.float32),
        grid=(n_tiles,),
        in_specs=[
            pl.BlockSpec((TB, S), lambda i: (i, 0)),
            pl.BlockSpec((S, H), lambda i: (0, 0)),
            pl.BlockSpec((1, H), lambda i: (0, 0)),
            pl.BlockSpec((H, A), lambda i: (0, 0)),
            pl.BlockSpec((1, A), lambda i: (0, 0)),
        ],
        out_specs=pl.BlockSpec((TB, A), lambda i: (i, 0)),
        compiler_params=pltpu.CompilerParams(
            dimension_semantics=("parallel",)),
    )(x_p, w1, b1, w2, b2)
    return mu_p if Bp == B else mu_p[:B]


def kernel(x, w1, b1, w2, b2, sigma_param, episode_number):
    mu = _forward(x, w1, b1, w2, b2)
    sigma = jnp.asarray(5.0, dtype=jnp.float32)
    return mu, sigma


# speedup vs baseline: 1.5737x; 1.1533x over previous
import functools

import jax
import jax.numpy as jnp
from jax.experimental import pallas as pl
from jax.experimental.pallas import tpu as pltpu


def _mlp_kernel(x_ref, w1_ref, b1_ref, w2_ref, b2_ref, mu_ref):
    xb = x_ref[...].astype(jnp.bfloat16)
    w1b = w1_ref[...].astype(jnp.bfloat16)
    h = jnp.dot(xb, w1b, preferred_element_type=jnp.float32)
    h = jnp.maximum(h + b1_ref[...], 0.0)
    w2b = w2_ref[...].astype(jnp.bfloat16)
    mu = jnp.dot(h.astype(jnp.bfloat16), w2b,
                 preferred_element_type=jnp.float32)
    mu_ref[...] = mu + b2_ref[...]


def _round_up(n, m):
    return ((n + m - 1) // m) * m


@functools.partial(jax.jit, static_argnames=("batch_tile",))
def _forward(x, w1, b1, w2, b2, batch_tile=8192):
    B, S = x.shape
    H = w1.shape[1]
    A = w2.shape[1]

    TB = min(batch_tile, _round_up(B, 8))
    Bp = _round_up(B, TB)
    x_p = x if Bp == B else jnp.pad(x, ((0, Bp - B), (0, 0)))
    n_tiles = Bp // TB

    mu_p = pl.pallas_call(
        _mlp_kernel,
        out_shape=jax.ShapeDtypeStruct((Bp, A), jnp.float32),
        grid=(n_tiles,),
        in_specs=[
            pl.BlockSpec((TB, S), lambda i: (i, 0)),
            pl.BlockSpec((S, H), lambda i: (0, 0)),
            pl.BlockSpec((1, H), lambda i: (0, 0)),
            pl.BlockSpec((H, A), lambda i: (0, 0)),
            pl.BlockSpec((1, A), lambda i: (0, 0)),
        ],
        out_specs=pl.BlockSpec((TB, A), lambda i: (i, 0)),
        compiler_params=pltpu.CompilerParams(
            dimension_semantics=("parallel",)),
    )(x_p, w1, b1, w2, b2)
    return mu_p if Bp == B else mu_p[:B]


def kernel(x, w1, b1, w2, b2, sigma_param, episode_number):
    mu = _forward(x, w1, b1, w2, b2)
    sigma = jnp.asarray(5.0, dtype=jnp.float32)
    return mu, sigma


# speedup vs baseline: 1.6148x; 1.0261x over previous
import functools

import jax
import jax.numpy as jnp
from jax.experimental import pallas as pl
from jax.experimental.pallas import tpu as pltpu


def _mlp_kernel(x_ref, w1_ref, b1_ref, w2_ref, b2_ref, mu_ref):
    xb = x_ref[...].astype(jnp.bfloat16)
    w1b = w1_ref[...].astype(jnp.bfloat16)
    h = jnp.dot(xb, w1b, preferred_element_type=jnp.float32)
    h = jnp.maximum(h + b1_ref[...], 0.0)
    w2b = w2_ref[...].astype(jnp.bfloat16)
    mu = jnp.dot(h.astype(jnp.bfloat16), w2b,
                 preferred_element_type=jnp.float32)
    mu_ref[...] = mu + b2_ref[...]


def _round_up(n, m):
    return ((n + m - 1) // m) * m


@functools.partial(jax.jit, static_argnames=("batch_tile",))
def _forward(x, w1, b1, w2, b2, batch_tile=16384):
    B, S = x.shape
    H = w1.shape[1]
    A = w2.shape[1]

    TB = min(batch_tile, _round_up(B, 8))
    Bp = _round_up(B, TB)
    x_p = x if Bp == B else jnp.pad(x, ((0, Bp - B), (0, 0)))
    n_tiles = Bp // TB

    mu_p = pl.pallas_call(
        _mlp_kernel,
        out_shape=jax.ShapeDtypeStruct((Bp, A), jnp.float32),
        grid=(n_tiles,),
        in_specs=[
            pl.BlockSpec((TB, S), lambda i: (i, 0)),
            pl.BlockSpec((S, H), lambda i: (0, 0)),
            pl.BlockSpec((1, H), lambda i: (0, 0)),
            pl.BlockSpec((H, A), lambda i: (0, 0)),
            pl.BlockSpec((1, A), lambda i: (0, 0)),
        ],
        out_specs=pl.BlockSpec((TB, A), lambda i: (i, 0)),
        compiler_params=pltpu.CompilerParams(
            dimension_semantics=("parallel",)),
    )(x_p, w1, b1, w2, b2)
    return mu_p if Bp == B else mu_p[:B]


def kernel(x, w1, b1, w2, b2, sigma_param, episode_number):
    mu = _forward(x, w1, b1, w2, b2)
    sigma = jnp.asarray(5.0, dtype=jnp.float32)
    return mu, sigma
